# Initial kernel scaffold; baseline (speedup 1.0000x reference)
import jax
import jax.numpy as jnp
from jax.experimental import pallas as pl


def kernel(xyz, cls_label_one_hot, params):
    raise NotImplementedError("write your pallas kernel here")


# speedup vs baseline: 5.7387x; 5.7387x over previous
import functools

import numpy as np
import jax
import jax.numpy as jnp
from jax.experimental import pallas as pl


_f32 = jnp.float32


def _fps(xnb, ynb, znb, npoint):
    N, B = xnb.shape

    def body(x_r, y_r, z_r, ox_r, oy_r, oz_r):
        x = x_r[...]
        y = y_r[...]
        z = z_r[...]
        iota = jax.lax.broadcasted_iota(jnp.int32, (N, B), 0)

        def step(t, carry):
            dist, far = carry
            eq = (iota == far).astype(_f32)
            cx = jnp.sum(eq * x, axis=0, keepdims=True)
            cy = jnp.sum(eq * y, axis=0, keepdims=True)
            cz = jnp.sum(eq * z, axis=0, keepdims=True)
            ox_r[pl.ds(t, 1), :] = cx
            oy_r[pl.ds(t, 1), :] = cy
            oz_r[pl.ds(t, 1), :] = cz
            dx = x - cx
            dy = y - cy
            dz = z - cz
            d = (dx * dx + dy * dy) + dz * dz
            dist = jnp.minimum(dist, d)
            m = jnp.max(dist, axis=0, keepdims=True)
            cand = jnp.where(dist == m, iota, N)
            far = jnp.min(cand, axis=0, keepdims=True)
            return dist, far

        dist0 = jnp.full((N, B), 1e10, _f32)
        far0 = jnp.zeros((1, B), jnp.int32)
        jax.lax.fori_loop(0, npoint, step, (dist0, far0))

    out = jax.ShapeDtypeStruct((npoint, B), _f32)
    return pl.pallas_call(body, out_shape=[out, out, out])(xnb, ynb, znb)


def _ball_group(px, py, pz, cx, cy, cz, table, r2, ns):
    B, N, _ = px.shape
    S = cx.shape[2]
    Cp = table.shape[2]

    def body(px_r, py_r, pz_r, cx_r, cy_r, cz_r, t_r, g_r):
        x = px_r[0]
        y = py_r[0]
        z = pz_r[0]
        a = cx_r[0]
        b = cy_r[0]
        c = cz_r[0]
        pp = (x * x + y * y) + z * z
        cc = (a * a + b * b) + c * c
        P = jnp.concatenate([x, y, z], 1)
        C3 = jnp.concatenate([a, b, c], 0)
        dot = jax.lax.dot_general(P, C3, (((1,), (0,)), ((), ())),
                                  preferred_element_type=_f32)
        d = (pp + cc) - 2.0 * dot
        iota = jax.lax.broadcasted_iota(jnp.int32, (N, S), 0)
        masked0 = jnp.where(d > r2, N, iota)
        ft = t_r[0]

        def step(k, carry):
            masked, first = carry
            cur = jnp.min(masked, axis=0, keepdims=True)
            first = jnp.where(k == 0, cur, first)
            curfix = jnp.where(cur == N, first, cur)
            gi = jnp.minimum(curfix, N - 1)
            eq = (iota == gi).astype(_f32)
            gk = jax.lax.dot_general(
                eq, ft, (((0,), (0,)), ((), ())),
                preferred_element_type=_f32,
                precision=jax.lax.Precision.HIGHEST)
            g_r[0, k] = gk
            masked = jnp.where(masked == cur, N, masked)
            return masked, first

        jax.lax.fori_loop(0, ns, step,
                          (masked0, jnp.zeros((1, S), jnp.int32)))

    return pl.pallas_call(
        body,
        grid=(B,),
        in_specs=[
            pl.BlockSpec((1, N, 1), lambda i: (i, 0, 0)),
            pl.BlockSpec((1, N, 1), lambda i: (i, 0, 0)),
            pl.BlockSpec((1, N, 1), lambda i: (i, 0, 0)),
            pl.BlockSpec((1, 1, S), lambda i: (i, 0, 0)),
            pl.BlockSpec((1, 1, S), lambda i: (i, 0, 0)),
            pl.BlockSpec((1, 1, S), lambda i: (i, 0, 0)),
            pl.BlockSpec((1, N, Cp), lambda i: (i, 0, 0)),
        ],
        out_specs=pl.BlockSpec((1, ns, S, Cp), lambda i: (i, 0, 0, 0)),
        out_shape=jax.ShapeDtypeStruct((B, ns, S, Cp), _f32),
    )(px, py, pz, cx, cy, cz, table)


def _const_spec(shape, ngrid):
    zeros = (0,) * len(shape)
    return pl.BlockSpec(shape, lambda *_: zeros)


def _bn_relu(Y, P):
    return jnp.maximum((Y - P[1:2, :]) / P[2:3, :] * P[3:4, :] + P[4:5, :],
                       0.0)


def _chain(Z, rest, k, last_affine):
    for i in range(k):
        W = rest[2 * i][...]
        P = rest[2 * i + 1]
        Y = jnp.dot(Z, W, preferred_element_type=_f32) + P[0:1, :]
        Z = _bn_relu(Y, P)
    W = rest[2 * k][...]
    P = rest[2 * k + 1]
    Y = jnp.dot(Z, W, preferred_element_type=_f32) + P[0:1, :]
    if last_affine:
        Y = _bn_relu(Y, P)
    return Y


def _bn_stack(base_inputs, base_specs, grid, prep, layers, Wfirst, R,
              pool=None, out_shape=None, out_spec=None):
    L = len(layers)
    Ws = [Wfirst] + [l[0] for l in layers[1:]]
    ngrid = len(grid)
    nbase = len(base_inputs)
    affines = []

    for k in range(L):
        Ck = Ws[k].shape[1]
        chain_in, chain_specs = [], []
        for i in range(k):
            chain_in += [Ws[i], affines[i]]
            chain_specs += [_const_spec(Ws[i].shape, ngrid),
                            _const_spec(affines[i].shape, ngrid)]
        bk = layers[k][1]
        chain_specs += [_const_spec(Ws[k].shape, ngrid),
                        _const_spec((8, Ck), ngrid)]

        def stats_body(*refs, _k=k):
            base = refs[:nbase]
            rest = refs[nbase:-1]
            out = refs[-1]
            Z = prep(*base)
            Y = _chain(Z, rest, _k, last_affine=False)
            first = sum(pl.program_id(i) for i in range(ngrid)) == 0

            @pl.when(first)
            def _():
                out[...] = jnp.zeros_like(out)

            Yc = Y - rest[2 * _k + 1][1:2, :]
            out[0:1, :] += jnp.sum(Y, axis=0, keepdims=True)
            out[1:2, :] += jnp.sum(Yc * Yc, axis=0, keepdims=True)

        def run_stats(center):
            Pk = jnp.concatenate([bk[None], center[None],
                                  jnp.zeros((6, Ck), _f32)], 0)
            return pl.pallas_call(
                stats_body,
                grid=grid,
                in_specs=base_specs + chain_specs,
                out_specs=_const_spec((8, Ck), ngrid),
                out_shape=jax.ShapeDtypeStruct((8, Ck), _f32),
            )(*(base_inputs + chain_in + [Ws[k], Pk]))

        mean = run_stats(jnp.zeros((Ck,), _f32))[0] / R
        var = run_stats(mean)[1] / R
        g = layers[k][2]
        bt = layers[k][3]
        denom = jnp.sqrt(var + 1e-5)
        affines.append(jnp.concatenate(
            [bk[None], mean[None], denom[None], g[None], bt[None],
             jnp.zeros((3, Ck), _f32)], 0))

    chain_in, chain_specs = [], []
    for i in range(L):
        chain_in += [Ws[i], affines[i]]
        chain_specs += [_const_spec(Ws[i].shape, ngrid),
                        _const_spec(affines[i].shape, ngrid)]
    CL = Ws[-1].shape[1]

    def final_body(*refs):
        base = refs[:nbase]
        rest = refs[nbase:-1]
        out = refs[-1]
        Z = prep(*base)
        Y = _chain(Z, rest, L - 1, last_affine=True)
        if pool is not None:
            ns, Sc = pool
            out[...] = jnp.max(Y.reshape(ns, Sc, CL), axis=0)[None]
        else:
            out[...] = Y

    return pl.pallas_call(
        final_body,
        grid=grid,
        in_specs=base_specs + chain_specs,
        out_specs=out_spec,
        out_shape=out_shape,
    )(*(base_inputs + chain_in))


def _stack_sa(G, c16, layers, Wfirst, ns, S, Sc, R):
    B = G.shape[0]
    Cp = G.shape[3]
    CL = layers[-1][0].shape[1]
    grid = (B, S // Sc)

    def prep(g_r, c_r):
        g = g_r[0].reshape(ns * Sc, Cp)
        c = c_r[0]
        cb = jnp.broadcast_to(c[None], (ns, Sc, Cp)).reshape(ns * Sc, Cp)
        return g - cb

    base_specs = [
        pl.BlockSpec((1, ns, Sc, Cp), lambda i, j: (i, 0, j, 0)),
        pl.BlockSpec((1, Sc, Cp), lambda i, j: (i, j, 0)),
    ]
    return _bn_stack(
        [G, c16], base_specs, grid, prep, layers, Wfirst, R,
        pool=(ns, Sc),
        out_shape=jax.ShapeDtypeStruct((B, S, CL), _f32),
        out_spec=pl.BlockSpec((1, Sc, CL), lambda i, j: (i, j, 0)))


def _stack_rows(X, layers, chunk):
    R, C0 = X.shape
    CL = layers[-1][0].shape[1]
    grid = (R // chunk,)

    def prep(x_r):
        return x_r[...]

    base_specs = [pl.BlockSpec((chunk, C0), lambda i: (i, 0))]
    return _bn_stack(
        [X], base_specs, grid, prep, layers, layers[0][0], R,
        pool=None,
        out_shape=jax.ShapeDtypeStruct((R, CL), _f32),
        out_spec=pl.BlockSpec((chunk, CL), lambda i: (i, 0)))


def _interp3(px, py, pz, cx, cy, cz, F):
    B, N1, _ = px.shape
    S2 = cx.shape[2]
    C = F.shape[2]

    def body(px_r, py_r, pz_r, cx_r, cy_r, cz_r, f_r, o_r):
        x = px_r[0]
        y = py_r[0]
        z = pz_r[0]
        a = cx_r[0]
        b = cy_r[0]
        c = cz_r[0]
        pp = (x * x + y * y) + z * z
        cc = (a * a + b * b) + c * c
        P = jnp.concatenate([x, y, z], 1)
        C3 = jnp.concatenate([a, b, c], 0)
        dot = jax.lax.dot_general(P, C3, (((1,), (0,)), ((), ())),
                                  preferred_element_type=_f32)
        d = (pp + cc) - 2.0 * dot
        ft = f_r[0]
        iota = jax.lax.broadcasted_iota(jnp.int32, (N1, S2), 1)
        gs, ws = [], []
        for _ in range(3):
            m = jnp.min(d, axis=1, keepdims=True)
            cand = jnp.where(d == m, iota, S2)
            ik = jnp.min(cand, axis=1, keepdims=True)
            sel = iota == ik
            eq = sel.astype(_f32)
            gs.append(jnp.dot(eq, ft, preferred_element_type=_f32,
                              precision=jax.lax.Precision.HIGHEST))
            ws.append(1.0 / (m + 1e-8))
            d = jnp.where(sel, jnp.float32(np.inf), d)
        wsum = (ws[0] + ws[1]) + ws[2]
        acc = gs[0] * (ws[0] / wsum)
        acc = acc + gs[1] * (ws[1] / wsum)
        acc = acc + gs[2] * (ws[2] / wsum)
        o_r[0] = acc

    return pl.pallas_call(
        body,
        grid=(B,),
        in_specs=[
            pl.BlockSpec((1, N1, 1), lambda i: (i, 0, 0)),
            pl.BlockSpec((1, N1, 1), lambda i: (i, 0, 0)),
            pl.BlockSpec((1, N1, 1), lambda i: (i, 0, 0)),
            pl.BlockSpec((1, 1, S2), lambda i: (i, 0, 0)),
            pl.BlockSpec((1, 1, S2), lambda i: (i, 0, 0)),
            pl.BlockSpec((1, 1, S2), lambda i: (i, 0, 0)),
            pl.BlockSpec((1, S2, C), lambda i: (i, 0, 0)),
        ],
        out_specs=pl.BlockSpec((1, N1, C), lambda i: (i, 0, 0)),
        out_shape=jax.ShapeDtypeStruct((B, N1, C), _f32),
    )(px, py, pz, cx, cy, cz, F)


def kernel(xyz, cls_label_one_hot, params):
    B, IC, N = xyz.shape
    CLS = cls_label_one_hot.shape[1]
    pts = jnp.transpose(xyz, (0, 2, 1))
    cls = jnp.transpose(cls_label_one_hot, (0, 2, 1))
    x0 = pts[..., 0]
    y0 = pts[..., 1]
    z0 = pts[..., 2]

    S1, NS1 = 512, 32
    ox1, oy1, oz1 = _fps(x0.T, y0.T, z0.T, S1)
    l1x, l1y, l1z = ox1.T, oy1.T, oz1.T
    table1 = jnp.concatenate(
        [pts[..., :3], pts, jnp.zeros((B, N, 4), _f32)], -1)
    G1 = _ball_group(
        x0[:, :, None], y0[:, :, None], z0[:, :, None],
        l1x[:, None, :], l1y[:, None, :], l1z[:, None, :],
        table1, np.float32(0.2 ** 2), NS1)
    c16_1 = jnp.concatenate(
        [jnp.stack([l1x, l1y, l1z], -1), jnp.zeros((B, S1, 13), _f32)], -1)
    W1a = jnp.zeros((16, 64), _f32).at[:12].set(params['sa1'][0][0])
    l1_points = _stack_sa(G1, c16_1, params['sa1'], W1a,
                          ns=NS1, S=S1, Sc=128, R=B * S1 * NS1)

    S2, NS2 = 128, 64
    ox2, oy2, oz2 = _fps(ox1, oy1, oz1, S2)
    l2x, l2y, l2z = ox2.T, oy2.T, oz2.T
    table2 = jnp.concatenate(
        [jnp.stack([l1x, l1y, l1z], -1), l1_points,
         jnp.zeros((B, S1, 13), _f32)], -1)
    G2 = _ball_group(
        l1x[:, :, None], l1y[:, :, None], l1z[:, :, None],
        l2x[:, None, :], l2y[:, None, :], l2z[:, None, :],
        table2, np.float32(0.4 ** 2), NS2)
    c144_2 = jnp.concatenate(
        [jnp.stack([l2x, l2y, l2z], -1), jnp.zeros((B, S2, 141), _f32)], -1)
    W2a = jnp.zeros((144, 128), _f32).at[:131].set(params['sa2'][0][0])
    l2_points = _stack_sa(G2, c144_2, params['sa2'], W2a,
                          ns=NS2, S=S2, Sc=64, R=B * S2 * NS2)

    X3 = jnp.concatenate(
        [jnp.stack([l2x, l2y, l2z], -1), l2_points], -1)
    G3 = X3[:, :, None, :]
    c3 = jnp.zeros((B, 1, 259), _f32)
    l3_points = _stack_sa(G3, c3, params['sa3'], params['sa3'][0][0],
                          ns=S2, S=1, Sc=1, R=B * S2)
    l3 = l3_points.reshape(B, 1024)

    Xfp3 = jnp.concatenate(
        [l2_points, jnp.broadcast_to(l3[:, None, :], (B, S2, 1024))],
        -1).reshape(B * S2, 1280)
    l2_new = _stack_rows(Xfp3, params['fp3'], chunk=S2)

    interp2 = _interp3(
        l1x[:, :, None], l1y[:, :, None], l1z[:, :, None],
        l2x[:, None, :], l2y[:, None, :], l2z[:, None, :],
        l2_new.reshape(B, S2, 256))
    Xfp2 = jnp.concatenate(
        [l1_points.reshape(B * S1, 128), interp2.reshape(B * S1, 256)], -1)
    l1_new = _stack_rows(Xfp2, params['fp2'], chunk=S1)

    interp1 = _interp3(
        x0[:, :, None], y0[:, :, None], z0[:, :, None],
        l1x[:, None, :], l1y[:, None, :], l1z[:, None, :],
        l1_new.reshape(B, S1, 128))
    Xfp1 = jnp.concatenate(
        [cls.reshape(B * N, CLS), pts.reshape(B * N, IC)[:, :3],
         pts.reshape(B * N, IC), interp1.reshape(B * N, 128)], -1)
    l0_new = _stack_rows(Xfp1, params['fp1'], chunk=N)

    return jnp.transpose(l0_new.reshape(B, N, 128), (0, 2, 1))
